# baseline (speedup 1.0000x reference)
import jax
import jax.numpy as jnp
from jax.experimental import pallas as pl
from jax.experimental.pallas import tpu as pltpu

N = 4096
N_IN = 512
N_H = 128

TM = 256


def _mega_kernel(adj_ref, s1_ref, s2_ref, w_ref, b_ref,
                 h1_ref, h2_ref, acc_ref, f_ref):
    i = pl.program_id(0)

    @pl.when(i == 0)
    def _():
        w = w_ref[...]
        f_ref[:, :N_H] = jnp.dot(s1_ref[...], w, preferred_element_type=jnp.float32)
        f_ref[:, N_H:] = jnp.dot(s2_ref[...], w, preferred_element_type=jnp.float32)

    agg = jnp.dot(adj_ref[...], f_ref[...], preferred_element_type=jnp.float32)
    b = b_ref[...]
    h1 = jnp.maximum(agg[:, :N_H] + b, 0.0)
    h2 = jnp.maximum(agg[:, N_H:] + b, 0.0)
    h1_ref[...] = h1.astype(jnp.bfloat16)
    h2_ref[...] = h2.astype(jnp.bfloat16)
    part = jnp.sum(h1, axis=0, keepdims=True)

    @pl.when(i == 0)
    def _():
        acc_ref[...] = part

    @pl.when(i != 0)
    def _():
        acc_ref[...] += part


def _fin_kernel(h1_ref, h2_ref, acc_ref, wb_ref, prompt_ref, bb_ref, o_ref):
    c = jax.nn.sigmoid(acc_ref[...] * (1.0 / N))
    v = jax.lax.dot_general(c, wb_ref[...], (((1,), (1,)), ((), ())),
                            preferred_element_type=jnp.float32)
    v2 = v * prompt_ref[...]
    bb = bb_ref[0, 0]
    h1 = h1_ref[...].astype(jnp.float32)
    h2 = h2_ref[...].astype(jnp.float32)
    sc1 = jax.lax.dot_general(v, h1, (((1,), (1,)), ((), ())),
                              preferred_element_type=jnp.float32)
    sc2 = jax.lax.dot_general(v2, h2, (((1,), (1,)), ((), ())),
                              preferred_element_type=jnp.float32)
    o_ref[0:1, :] = sc1 + bb
    o_ref[1:2, :] = sc2 + bb


def kernel(seq1, seq2, adj, sparse, W_gcn, b_gcn, prompt, W_bil, b_bil):
    s1 = seq1[0]
    s2 = seq2[0]
    a = adj[0]
    b2 = b_gcn.reshape(1, N_H)
    bb = b_bil.reshape(1, 1)

    h1, h2, acc = pl.pallas_call(
        _mega_kernel,
        grid=(N // TM,),
        in_specs=[
            pl.BlockSpec((TM, N), lambda i: (i, 0)),
            pl.BlockSpec((N, N_IN), lambda i: (0, 0)),
            pl.BlockSpec((N, N_IN), lambda i: (0, 0)),
            pl.BlockSpec((N_IN, N_H), lambda i: (0, 0)),
            pl.BlockSpec((1, N_H), lambda i: (0, 0)),
        ],
        out_specs=[
            pl.BlockSpec((TM, N_H), lambda i: (i, 0)),
            pl.BlockSpec((TM, N_H), lambda i: (i, 0)),
            pl.BlockSpec((1, N_H), lambda i: (0, 0)),
        ],
        out_shape=[
            jax.ShapeDtypeStruct((N, N_H), jnp.bfloat16),
            jax.ShapeDtypeStruct((N, N_H), jnp.bfloat16),
            jax.ShapeDtypeStruct((1, N_H), jnp.float32),
        ],
        scratch_shapes=[pltpu.VMEM((N, 2 * N_H), jnp.float32)],
    )(a, s1, s2, W_gcn, b2)

    out = pl.pallas_call(
        _fin_kernel,
        in_specs=[
            pl.BlockSpec((N, N_H), lambda: (0, 0)),
            pl.BlockSpec((N, N_H), lambda: (0, 0)),
            pl.BlockSpec((1, N_H), lambda: (0, 0)),
            pl.BlockSpec((N_H, N_H), lambda: (0, 0)),
            pl.BlockSpec((1, N_H), lambda: (0, 0)),
            pl.BlockSpec((1, 1), lambda: (0, 0)),
        ],
        out_specs=pl.BlockSpec((2, N), lambda: (0, 0)),
        out_shape=jax.ShapeDtypeStruct((2, N), jnp.float32),
    )(h1, h2, acc, W_bil, prompt, bb)

    return out.reshape(1, 2 * N)


# speedup vs baseline: 1.1056x; 1.1056x over previous
import jax
import jax.numpy as jnp
from jax.experimental import pallas as pl
from jax.experimental.pallas import tpu as pltpu

N = 4096
N_IN = 512
N_H = 128

TM = 1024


def _mega_kernel(adj_ref, s1_ref, s2_ref, w_ref, b_ref,
                 h1_ref, h2_ref, acc_ref, f_ref):
    i = pl.program_id(0)

    @pl.when(i == 0)
    def _():
        w = w_ref[...]
        f_ref[:, :N_H] = jnp.dot(s1_ref[...], w, preferred_element_type=jnp.float32)
        f_ref[:, N_H:] = jnp.dot(s2_ref[...], w, preferred_element_type=jnp.float32)

    agg = jnp.dot(adj_ref[...], f_ref[...], preferred_element_type=jnp.float32)
    b = b_ref[...]
    h1 = jnp.maximum(agg[:, :N_H] + b, 0.0)
    h2 = jnp.maximum(agg[:, N_H:] + b, 0.0)
    h1_ref[...] = h1.astype(jnp.bfloat16)
    h2_ref[...] = h2.astype(jnp.bfloat16)
    part = jnp.sum(h1, axis=0, keepdims=True)

    @pl.when(i == 0)
    def _():
        acc_ref[...] = part

    @pl.when(i != 0)
    def _():
        acc_ref[...] += part


def _fin_kernel(h1_ref, h2_ref, acc_ref, wb_ref, prompt_ref, bb_ref, o_ref):
    c = jax.nn.sigmoid(acc_ref[...] * (1.0 / N))
    v = jax.lax.dot_general(c, wb_ref[...], (((1,), (1,)), ((), ())),
                            preferred_element_type=jnp.float32)
    v2 = v * prompt_ref[...]
    bb = bb_ref[0, 0]
    h1 = h1_ref[...].astype(jnp.float32)
    h2 = h2_ref[...].astype(jnp.float32)
    sc1 = jax.lax.dot_general(v, h1, (((1,), (1,)), ((), ())),
                              preferred_element_type=jnp.float32)
    sc2 = jax.lax.dot_general(v2, h2, (((1,), (1,)), ((), ())),
                              preferred_element_type=jnp.float32)
    o_ref[0:1, :] = sc1 + bb
    o_ref[1:2, :] = sc2 + bb


def kernel(seq1, seq2, adj, sparse, W_gcn, b_gcn, prompt, W_bil, b_bil):
    s1 = seq1[0]
    s2 = seq2[0]
    a = adj[0]
    b2 = b_gcn.reshape(1, N_H)
    bb = b_bil.reshape(1, 1)

    h1, h2, acc = pl.pallas_call(
        _mega_kernel,
        grid=(N // TM,),
        in_specs=[
            pl.BlockSpec((TM, N), lambda i: (i, 0)),
            pl.BlockSpec((N, N_IN), lambda i: (0, 0)),
            pl.BlockSpec((N, N_IN), lambda i: (0, 0)),
            pl.BlockSpec((N_IN, N_H), lambda i: (0, 0)),
            pl.BlockSpec((1, N_H), lambda i: (0, 0)),
        ],
        out_specs=[
            pl.BlockSpec((TM, N_H), lambda i: (i, 0)),
            pl.BlockSpec((TM, N_H), lambda i: (i, 0)),
            pl.BlockSpec((1, N_H), lambda i: (0, 0)),
        ],
        out_shape=[
            jax.ShapeDtypeStruct((N, N_H), jnp.bfloat16),
            jax.ShapeDtypeStruct((N, N_H), jnp.bfloat16),
            jax.ShapeDtypeStruct((1, N_H), jnp.float32),
        ],
        scratch_shapes=[pltpu.VMEM((N, 2 * N_H), jnp.float32)],
    )(a, s1, s2, W_gcn, b2)

    out = pl.pallas_call(
        _fin_kernel,
        in_specs=[
            pl.BlockSpec((N, N_H), lambda: (0, 0)),
            pl.BlockSpec((N, N_H), lambda: (0, 0)),
            pl.BlockSpec((1, N_H), lambda: (0, 0)),
            pl.BlockSpec((N_H, N_H), lambda: (0, 0)),
            pl.BlockSpec((1, N_H), lambda: (0, 0)),
            pl.BlockSpec((1, 1), lambda: (0, 0)),
        ],
        out_specs=pl.BlockSpec((2, N), lambda: (0, 0)),
        out_shape=jax.ShapeDtypeStruct((2, N), jnp.float32),
    )(h1, h2, acc, W_bil, prompt, bb)

    return out.reshape(1, 2 * N)


# speedup vs baseline: 1.1062x; 1.0005x over previous
import jax
import jax.numpy as jnp
from jax.experimental import pallas as pl
from jax.experimental.pallas import tpu as pltpu

N = 4096
N_IN = 512
N_H = 128

TM = 512
NH2 = N // 2


def _mega_kernel(adjl_ref, adjr_ref, s1_ref, s2_ref, w_ref, b_ref,
                 h1_ref, h2_ref, acc_ref, f_ref):
    i = pl.program_id(0)

    @pl.when(i == 0)
    def _():
        w = w_ref[...]
        f_ref[:, :N_H] = jnp.dot(s1_ref[...], w, preferred_element_type=jnp.float32)
        f_ref[:, N_H:] = jnp.dot(s2_ref[...], w, preferred_element_type=jnp.float32)

    agg = jnp.dot(adjl_ref[...], f_ref[:NH2, :], preferred_element_type=jnp.float32)
    agg = agg + jnp.dot(adjr_ref[...], f_ref[NH2:, :], preferred_element_type=jnp.float32)
    b = b_ref[...]
    h1 = jnp.maximum(agg[:, :N_H] + b, 0.0)
    h2 = jnp.maximum(agg[:, N_H:] + b, 0.0)
    h1_ref[...] = h1.astype(jnp.bfloat16)
    h2_ref[...] = h2.astype(jnp.bfloat16)
    part = jnp.sum(h1, axis=0, keepdims=True)

    @pl.when(i == 0)
    def _():
        acc_ref[...] = part

    @pl.when(i != 0)
    def _():
        acc_ref[...] += part


def _fin_kernel(h1_ref, h2_ref, acc_ref, wb_ref, prompt_ref, bb_ref, o_ref):
    c = jax.nn.sigmoid(acc_ref[...] * (1.0 / N))
    v = jax.lax.dot_general(c, wb_ref[...], (((1,), (1,)), ((), ())),
                            preferred_element_type=jnp.float32)
    v2 = v * prompt_ref[...]
    bb = bb_ref[0, 0]
    h1 = h1_ref[...].astype(jnp.float32)
    h2 = h2_ref[...].astype(jnp.float32)
    sc1 = jax.lax.dot_general(v, h1, (((1,), (1,)), ((), ())),
                              preferred_element_type=jnp.float32)
    sc2 = jax.lax.dot_general(v2, h2, (((1,), (1,)), ((), ())),
                              preferred_element_type=jnp.float32)
    o_ref[0:1, :] = sc1 + bb
    o_ref[1:2, :] = sc2 + bb


def kernel(seq1, seq2, adj, sparse, W_gcn, b_gcn, prompt, W_bil, b_bil):
    s1 = seq1[0]
    s2 = seq2[0]
    a = adj[0]
    b2 = b_gcn.reshape(1, N_H)
    bb = b_bil.reshape(1, 1)

    h1, h2, acc = pl.pallas_call(
        _mega_kernel,
        grid=(N // TM,),
        in_specs=[
            pl.BlockSpec((TM, NH2), lambda i: (i, 0)),
            pl.BlockSpec((TM, NH2), lambda i: (i, 1)),
            pl.BlockSpec((N, N_IN), lambda i: (0, 0)),
            pl.BlockSpec((N, N_IN), lambda i: (0, 0)),
            pl.BlockSpec((N_IN, N_H), lambda i: (0, 0)),
            pl.BlockSpec((1, N_H), lambda i: (0, 0)),
        ],
        out_specs=[
            pl.BlockSpec((TM, N_H), lambda i: (i, 0)),
            pl.BlockSpec((TM, N_H), lambda i: (i, 0)),
            pl.BlockSpec((1, N_H), lambda i: (0, 0)),
        ],
        out_shape=[
            jax.ShapeDtypeStruct((N, N_H), jnp.bfloat16),
            jax.ShapeDtypeStruct((N, N_H), jnp.bfloat16),
            jax.ShapeDtypeStruct((1, N_H), jnp.float32),
        ],
        scratch_shapes=[pltpu.VMEM((N, 2 * N_H), jnp.float32)],
    )(a, a, s1, s2, W_gcn, b2)

    out = pl.pallas_call(
        _fin_kernel,
        in_specs=[
            pl.BlockSpec((N, N_H), lambda: (0, 0)),
            pl.BlockSpec((N, N_H), lambda: (0, 0)),
            pl.BlockSpec((1, N_H), lambda: (0, 0)),
            pl.BlockSpec((N_H, N_H), lambda: (0, 0)),
            pl.BlockSpec((1, N_H), lambda: (0, 0)),
            pl.BlockSpec((1, 1), lambda: (0, 0)),
        ],
        out_specs=pl.BlockSpec((2, N), lambda: (0, 0)),
        out_shape=jax.ShapeDtypeStruct((2, N), jnp.float32),
    )(h1, h2, acc, W_bil, prompt, bb)

    return out.reshape(1, 2 * N)
